# baseline (speedup 1.0000x reference)
import functools
import math

import jax
import jax.numpy as jnp
from jax.experimental import pallas as pl
from jax.experimental.pallas import tpu as pltpu

B = 4096
C = 3
H = 32
W = 32
E = 16
D = 32
C1 = 16
C2 = 32
TB = 128
GW = 8
NG = W // GW


def _router_body(xin_ref, tid_ref, m1_ref, m2_ref, b1t_ref, b2t_ref,
                 tw1_ref, tb1_ref, tw2_ref, tb2_ref, dw_ref, db_ref,
                 ekt_ref, gw_ref, gb_ref, gates_ref, load_ref,
                 xp_ref, x2p_ref):
    f32 = jnp.float32

    @pl.when(pl.program_id(0) == 0)
    def _():
        xp_ref[...] = jnp.zeros_like(xp_ref)
        x2p_ref[...] = jnp.zeros_like(x2p_ref)

    x = xin_ref[...]
    for ci in range(C):
        xp_ref[:, 1:H + 1, (W + 2) * ci + 1:(W + 2) * ci + W + 1] = \
            x[:, :, W * ci:W * ci + W].astype(jnp.bfloat16)
    xpb = xp_ref[...]
    acc1 = jnp.zeros((TB * H, W * C1), dtype=f32)
    for dy in range(3):
        lhs = xpb[:, dy:dy + H, :].reshape(TB * H, C * (W + 2))
        acc1 = acc1 + jnp.dot(lhs, m1_ref[dy],
                              preferred_element_type=f32)
    x2 = jnp.maximum(acc1 + b1t_ref[...], 0.0)
    x2p_ref[:, 1:H + 1, C1:C1 * (W + 1)] = \
        x2.reshape(TB, H, W * C1).astype(jnp.bfloat16)
    x2p = x2p_ref[...]
    pool = jnp.zeros((TB, C2), dtype=f32)
    for g in range(NG):
        acc2 = jnp.zeros((TB * H, GW * C2), dtype=f32)
        for dy in range(3):
            lhs = x2p[:, dy:dy + H, C1 * GW * g:C1 * (GW * g + GW + 2)]
            acc2 = acc2 + jnp.dot(lhs.reshape(TB * H, (GW + 2) * C1),
                                  m2_ref[dy], preferred_element_type=f32)
        yg = jnp.maximum(acc2 + b2t_ref[...], 0.0)
        yg = jnp.sum(yg.reshape(TB, H, GW * C2), axis=1)
        for j in range(GW):
            pool = pool + yg[:, C2 * j:C2 * j + C2]
    pooled = pool * (1.0 / (H * W))
    deg_embed = jnp.dot(pooled, dw_ref[...], preferred_element_type=f32) \
        + db_ref[...]
    t = tid_ref[...]
    h = jnp.maximum(t * tw1_ref[...] + tb1_ref[...], 0.0)
    task_embed = jnp.dot(h, tw2_ref[...], preferred_element_type=f32) \
        + tb2_ref[...]
    combined = task_embed + 0.2 * deg_embed
    sim = jnp.dot(combined, ekt_ref[...], preferred_element_type=f32) \
        * (1.0 / math.sqrt(D))
    sim = sim - jnp.max(sim, axis=1, keepdims=True)
    ew = jnp.exp(sim)
    ew = ew / jnp.sum(ew, axis=1, keepdims=True)
    logits = jnp.dot(ew, gw_ref[...], preferred_element_type=f32) \
        + gb_ref[...]
    iota = jax.lax.broadcasted_iota(jnp.int32, (TB, E), 1)
    m1 = jnp.max(logits, axis=1, keepdims=True)
    i1 = jnp.min(jnp.where(logits == m1, iota, E), axis=1, keepdims=True)
    oh1 = iota == i1
    masked = jnp.where(oh1, -1e30, logits)
    m2 = jnp.max(masked, axis=1, keepdims=True)
    i2 = jnp.min(jnp.where(masked == m2, iota, E), axis=1, keepdims=True)
    oh2 = iota == i2
    g1 = 1.0 / (1.0 + jnp.exp(m2 - m1))
    g2 = 1.0 / (1.0 + jnp.exp(m1 - m2))
    gates = jnp.where(oh1, g1, 0.0) + jnp.where(oh2, g2, 0.0)
    gates_ref[...] = gates
    part = jnp.sum(gates, axis=0, keepdims=True)

    @pl.when(pl.program_id(0) == 0)
    def _():
        load_ref[...] = jnp.zeros_like(load_ref)
    load_ref[...] += part


def kernel(task_id, degradation_info, train, task_w1, task_b1, task_w2,
           task_b2, conv_w1, conv_b1, conv_w2, conv_b2, deg_w, deg_b,
           expert_keys, gate_w, gate_b, noise_w, noise_b):
    del train, noise_w, noise_b
    f32 = jnp.float32
    xin = degradation_info.transpose(0, 2, 1, 3).reshape(B, H, C * W)
    tid = task_id.astype(f32)[:, None]

    xo1 = jnp.arange(W)
    kk1 = (jnp.arange(C)[:, None, None, None] * (W + 2)
           + xo1[None, :, None, None] + jnp.arange(3)[None, None, :, None])
    nn1 = xo1[None, :, None, None] * C1 + jnp.arange(C1)[None, None, None, :]
    vv1 = conv_w1.transpose(2, 1, 3, 0)[:, :, None, :, :]
    m1 = jnp.zeros((3, C * (W + 2), W * C1), dtype=f32)
    m1 = m1.at[:, kk1, nn1].set(
        jnp.broadcast_to(vv1, (3, C, W, 3, C1))).astype(jnp.bfloat16)

    xo2 = jnp.arange(GW)
    kk2 = ((xo2[:, None, None, None] + jnp.arange(3)[None, :, None, None]) * C1
           + jnp.arange(C1)[None, None, :, None])
    nn2 = (xo2[:, None, None, None] * C2
           + jnp.arange(C2)[None, None, None, :])
    vv2 = conv_w2.transpose(2, 3, 1, 0)[:, None, :, :, :]
    m2 = jnp.zeros((3, (GW + 2) * C1, GW * C2), dtype=f32)
    m2 = m2.at[:, kk2, nn2].set(
        jnp.broadcast_to(vv2, (3, GW, 3, C1, C2))).astype(jnp.bfloat16)

    b1t = jnp.tile(conv_b1, W)[None, :]
    b2t = jnp.tile(conv_b2, GW)[None, :]
    whole = lambda s: pl.BlockSpec(s, lambda i: (0,) * len(s))
    grid = (B // TB,)
    gates, load = pl.pallas_call(
        _router_body,
        grid=grid,
        in_specs=[
            pl.BlockSpec((TB, H, C * W), lambda i: (i, 0, 0)),
            pl.BlockSpec((TB, 1), lambda i: (i, 0)),
            whole((3, C * (W + 2), W * C1)),
            whole((3, (GW + 2) * C1, GW * C2)),
            whole((1, W * C1)),
            whole((1, GW * C2)),
            whole((1, D)), whole((1, D)), whole((D, D)), whole((1, D)),
            whole((D, D)), whole((1, D)), whole((D, E)), whole((E, E)),
            whole((1, E)),
        ],
        out_specs=[
            pl.BlockSpec((TB, E), lambda i: (i, 0)),
            pl.BlockSpec((1, E), lambda i: (0, 0)),
        ],
        out_shape=[
            jax.ShapeDtypeStruct((B, E), f32),
            jax.ShapeDtypeStruct((1, E), f32),
        ],
        scratch_shapes=[
            pltpu.VMEM((TB, H + 2, C * (W + 2)), jnp.bfloat16),
            pltpu.VMEM((TB, H + 2, (W + 2) * C1), jnp.bfloat16),
        ],
    )(xin, tid, m1, m2, b1t, b2t,
      task_w1, task_b1[None, :], task_w2, task_b2[None, :],
      deg_w, deg_b[None, :], expert_keys.T, gate_w, gate_b[None, :])
    return gates, load.reshape(E)


# speedup vs baseline: 1.0400x; 1.0400x over previous
import functools
import math

import jax
import jax.numpy as jnp
from jax.experimental import pallas as pl
from jax.experimental.pallas import tpu as pltpu

B = 4096
C = 3
H = 32
W = 32
E = 16
D = 32
C1 = 16
C2 = 32
TB = 128
GW = 8
NG = W // GW


def _router_body(xin_ref, tid_ref, m1_ref, m2_ref, ab_ref, b1t_ref, b2t_ref,
                 tw1_ref, tb1_ref, tw2_ref, tb2_ref, dw_ref, db_ref,
                 ekt_ref, gw_ref, gb_ref, gates_ref, load_ref,
                 xp_ref, x2p_ref):
    f32 = jnp.float32

    @pl.when(pl.program_id(0) == 0)
    def _():
        xp_ref[...] = jnp.zeros_like(xp_ref)
        x2p_ref[...] = jnp.zeros_like(x2p_ref)

    x = xin_ref[...]
    for ci in range(C):
        xp_ref[:, 1:H + 1, (W + 2) * ci + 1:(W + 2) * ci + W + 1] = \
            x[:, :, W * ci:W * ci + W].astype(jnp.bfloat16)
    xpb = xp_ref[...]
    acc1 = jnp.zeros((TB * H, W * C1), dtype=f32)
    for dy in range(3):
        lhs = xpb[:, dy:dy + H, :].reshape(TB * H, C * (W + 2))
        acc1 = acc1 + jnp.dot(lhs, m1_ref[dy],
                              preferred_element_type=f32)
    x2 = jnp.maximum(acc1 + b1t_ref[...], 0.0)
    x2p_ref[:, 1:H + 1, C1:C1 * (W + 1)] = \
        x2.reshape(TB, H, W * C1).astype(jnp.bfloat16)
    x2p = x2p_ref[...]
    sacc = jnp.zeros((TB * H, GW * C2), dtype=f32)
    for g in range(NG):
        acc2 = jnp.zeros((TB * H, GW * C2), dtype=f32)
        for dy in range(3):
            lhs = x2p[:, dy:dy + H, C1 * GW * g:C1 * (GW * g + GW + 2)]
            acc2 = acc2 + jnp.dot(lhs.reshape(TB * H, (GW + 2) * C1),
                                  m2_ref[dy], preferred_element_type=f32)
        sacc = sacc + jnp.maximum(acc2 + b2t_ref[...], 0.0)
    t_pool = jnp.dot(ab_ref[...], sacc.astype(jnp.bfloat16),
                     preferred_element_type=f32)
    pool = jnp.zeros((TB, C2), dtype=f32)
    for j in range(GW):
        pool = pool + t_pool[:, C2 * j:C2 * j + C2]
    pooled = pool * (1.0 / (H * W))
    deg_embed = jnp.dot(pooled, dw_ref[...], preferred_element_type=f32) \
        + db_ref[...]
    t = tid_ref[...]
    h = jnp.maximum(t * tw1_ref[...] + tb1_ref[...], 0.0)
    task_embed = jnp.dot(h, tw2_ref[...], preferred_element_type=f32) \
        + tb2_ref[...]
    combined = task_embed + 0.2 * deg_embed
    sim = jnp.dot(combined, ekt_ref[...], preferred_element_type=f32) \
        * (1.0 / math.sqrt(D))
    sim = sim - jnp.max(sim, axis=1, keepdims=True)
    ew = jnp.exp(sim)
    ew = ew / jnp.sum(ew, axis=1, keepdims=True)
    logits = jnp.dot(ew, gw_ref[...], preferred_element_type=f32) \
        + gb_ref[...]
    iota = jax.lax.broadcasted_iota(jnp.int32, (TB, E), 1)
    m1 = jnp.max(logits, axis=1, keepdims=True)
    i1 = jnp.min(jnp.where(logits == m1, iota, E), axis=1, keepdims=True)
    oh1 = iota == i1
    masked = jnp.where(oh1, -1e30, logits)
    m2 = jnp.max(masked, axis=1, keepdims=True)
    i2 = jnp.min(jnp.where(masked == m2, iota, E), axis=1, keepdims=True)
    oh2 = iota == i2
    g1 = 1.0 / (1.0 + jnp.exp(m2 - m1))
    g2 = 1.0 / (1.0 + jnp.exp(m1 - m2))
    gates = jnp.where(oh1, g1, 0.0) + jnp.where(oh2, g2, 0.0)
    gates_ref[...] = gates
    part = jnp.sum(gates, axis=0, keepdims=True)

    @pl.when(pl.program_id(0) == 0)
    def _():
        load_ref[...] = jnp.zeros_like(load_ref)
    load_ref[...] += part


def kernel(task_id, degradation_info, train, task_w1, task_b1, task_w2,
           task_b2, conv_w1, conv_b1, conv_w2, conv_b2, deg_w, deg_b,
           expert_keys, gate_w, gate_b, noise_w, noise_b):
    del train, noise_w, noise_b
    f32 = jnp.float32
    xin = degradation_info.transpose(0, 2, 1, 3).reshape(B, H, C * W)
    tid = task_id.astype(f32)[:, None]

    xo1 = jnp.arange(W)
    kk1 = (jnp.arange(C)[:, None, None, None] * (W + 2)
           + xo1[None, :, None, None] + jnp.arange(3)[None, None, :, None])
    nn1 = xo1[None, :, None, None] * C1 + jnp.arange(C1)[None, None, None, :]
    vv1 = conv_w1.transpose(2, 1, 3, 0)[:, :, None, :, :]
    m1 = jnp.zeros((3, C * (W + 2), W * C1), dtype=f32)
    m1 = m1.at[:, kk1, nn1].set(
        jnp.broadcast_to(vv1, (3, C, W, 3, C1))).astype(jnp.bfloat16)

    xo2 = jnp.arange(GW)
    kk2 = ((xo2[:, None, None, None] + jnp.arange(3)[None, :, None, None]) * C1
           + jnp.arange(C1)[None, None, :, None])
    nn2 = (xo2[:, None, None, None] * C2
           + jnp.arange(C2)[None, None, None, :])
    vv2 = conv_w2.transpose(2, 3, 1, 0)[:, None, :, :, :]
    m2 = jnp.zeros((3, (GW + 2) * C1, GW * C2), dtype=f32)
    m2 = m2.at[:, kk2, nn2].set(
        jnp.broadcast_to(vv2, (3, GW, 3, C1, C2))).astype(jnp.bfloat16)

    b1t = jnp.tile(conv_b1, W)[None, :]
    b2t = jnp.tile(conv_b2, GW)[None, :]
    ab = (jnp.arange(TB)[:, None] == jnp.arange(TB * H)[None, :] // H
          ).astype(jnp.bfloat16)
    whole = lambda s: pl.BlockSpec(s, lambda i: (0,) * len(s))
    grid = (B // TB,)
    gates, load = pl.pallas_call(
        _router_body,
        grid=grid,
        in_specs=[
            pl.BlockSpec((TB, H, C * W), lambda i: (i, 0, 0)),
            pl.BlockSpec((TB, 1), lambda i: (i, 0)),
            whole((3, C * (W + 2), W * C1)),
            whole((3, (GW + 2) * C1, GW * C2)),
            whole((TB, TB * H)),
            whole((1, W * C1)),
            whole((1, GW * C2)),
            whole((1, D)), whole((1, D)), whole((D, D)), whole((1, D)),
            whole((D, D)), whole((1, D)), whole((D, E)), whole((E, E)),
            whole((1, E)),
        ],
        out_specs=[
            pl.BlockSpec((TB, E), lambda i: (i, 0)),
            pl.BlockSpec((1, E), lambda i: (0, 0)),
        ],
        out_shape=[
            jax.ShapeDtypeStruct((B, E), f32),
            jax.ShapeDtypeStruct((1, E), f32),
        ],
        scratch_shapes=[
            pltpu.VMEM((TB, H + 2, C * (W + 2)), jnp.bfloat16),
            pltpu.VMEM((TB, H + 2, (W + 2) * C1), jnp.bfloat16),
        ],
    )(xin, tid, m1, m2, ab, b1t, b2t,
      task_w1, task_b1[None, :], task_w2, task_b2[None, :],
      deg_w, deg_b[None, :], expert_keys.T, gate_w, gate_b[None, :])
    return gates, load.reshape(E)


# speedup vs baseline: 1.0565x; 1.0159x over previous
import functools
import math

import jax
import jax.numpy as jnp
from jax.experimental import pallas as pl
from jax.experimental.pallas import tpu as pltpu

B = 4096
C = 3
H = 32
W = 32
E = 16
D = 32
C1 = 16
C2 = 32
TB = 128
GW = 8
NG = W // GW


def _router_body(xin_ref, tid_ref, m1_ref, m2_ref, b1t_ref, b2t_ref,
                 tw1_ref, tb1_ref, tw2_ref, tb2_ref, dw_ref, db_ref,
                 ekt_ref, gw_ref, gb_ref, gates_ref, load_ref,
                 xp_ref, x2p_ref):
    f32 = jnp.float32

    @pl.when(pl.program_id(0) == 0)
    def _():
        xp_ref[...] = jnp.zeros_like(xp_ref)
        x2p_ref[...] = jnp.zeros_like(x2p_ref)

    x = xin_ref[...]
    for ci in range(C):
        lo = (W + 2) * ci + 1
        xb = x[:, :, W * ci:W * ci + W].astype(jnp.bfloat16)
        xp_ref[0, :, 1:H, lo:lo + W] = xb[:, 0:H - 1]
        xp_ref[1, :, :, lo:lo + W] = xb
        xp_ref[2, :, 0:H - 1, lo:lo + W] = xb[:, 1:H]
    acc1 = jnp.zeros((TB * H, W * C1), dtype=f32)
    for dy in range(3):
        lhs = xp_ref[dy].reshape(TB * H, C * (W + 2))
        acc1 = acc1 + jnp.dot(lhs, m1_ref[dy],
                              preferred_element_type=f32)
    x2 = jnp.maximum(acc1 + b1t_ref[...], 0.0)
    x2r = x2.reshape(TB, H, W * C1).astype(jnp.bfloat16)
    x2p_ref[0, :, 1:H, C1:C1 * (W + 1)] = x2r[:, 0:H - 1]
    x2p_ref[1, :, :, C1:C1 * (W + 1)] = x2r
    x2p_ref[2, :, 0:H - 1, C1:C1 * (W + 1)] = x2r[:, 1:H]
    sacc = jnp.zeros((TB * H, GW * C2), dtype=f32)
    for g in range(NG):
        acc2 = jnp.zeros((TB * H, GW * C2), dtype=f32)
        for dy in range(3):
            lhs = x2p_ref[dy, :, :, C1 * GW * g:C1 * (GW * g + GW + 2)]
            acc2 = acc2 + jnp.dot(lhs.reshape(TB * H, (GW + 2) * C1),
                                  m2_ref[dy], preferred_element_type=f32)
        sacc = sacc + jnp.maximum(acc2 + b2t_ref[...], 0.0)
    s3 = sacc.reshape(TB, H, GW * C2)
    s3 = s3[:, 0:16, :] + s3[:, 16:32, :]
    s3 = s3[:, 0:8, :] + s3[:, 8:16, :]
    t_pool = jnp.sum(s3, axis=1)
    pool = jnp.zeros((TB, C2), dtype=f32)
    for j in range(GW):
        pool = pool + t_pool[:, C2 * j:C2 * j + C2]
    pooled = pool * (1.0 / (H * W))
    deg_embed = jnp.dot(pooled, dw_ref[...], preferred_element_type=f32) \
        + db_ref[...]
    t = tid_ref[...]
    h = jnp.maximum(t * tw1_ref[...] + tb1_ref[...], 0.0)
    task_embed = jnp.dot(h, tw2_ref[...], preferred_element_type=f32) \
        + tb2_ref[...]
    combined = task_embed + 0.2 * deg_embed
    sim = jnp.dot(combined, ekt_ref[...], preferred_element_type=f32) \
        * (1.0 / math.sqrt(D))
    sim = sim - jnp.max(sim, axis=1, keepdims=True)
    ew = jnp.exp(sim)
    ew = ew / jnp.sum(ew, axis=1, keepdims=True)
    logits = jnp.dot(ew, gw_ref[...], preferred_element_type=f32) \
        + gb_ref[...]
    iota = jax.lax.broadcasted_iota(jnp.int32, (TB, E), 1)
    m1 = jnp.max(logits, axis=1, keepdims=True)
    i1 = jnp.min(jnp.where(logits == m1, iota, E), axis=1, keepdims=True)
    oh1 = iota == i1
    masked = jnp.where(oh1, -1e30, logits)
    m2 = jnp.max(masked, axis=1, keepdims=True)
    i2 = jnp.min(jnp.where(masked == m2, iota, E), axis=1, keepdims=True)
    oh2 = iota == i2
    g1 = 1.0 / (1.0 + jnp.exp(m2 - m1))
    g2 = 1.0 / (1.0 + jnp.exp(m1 - m2))
    gates = jnp.where(oh1, g1, 0.0) + jnp.where(oh2, g2, 0.0)
    gates_ref[...] = gates
    part = jnp.sum(gates, axis=0, keepdims=True)

    @pl.when(pl.program_id(0) == 0)
    def _():
        load_ref[...] = jnp.zeros_like(load_ref)
    load_ref[...] += part


def kernel(task_id, degradation_info, train, task_w1, task_b1, task_w2,
           task_b2, conv_w1, conv_b1, conv_w2, conv_b2, deg_w, deg_b,
           expert_keys, gate_w, gate_b, noise_w, noise_b):
    del train, noise_w, noise_b
    f32 = jnp.float32
    xin = degradation_info.transpose(0, 2, 1, 3).reshape(B, H, C * W)
    tid = task_id.astype(f32)[:, None]

    xo1 = jnp.arange(W)
    kk1 = (jnp.arange(C)[:, None, None, None] * (W + 2)
           + xo1[None, :, None, None] + jnp.arange(3)[None, None, :, None])
    nn1 = xo1[None, :, None, None] * C1 + jnp.arange(C1)[None, None, None, :]
    vv1 = conv_w1.transpose(2, 1, 3, 0)[:, :, None, :, :]
    m1 = jnp.zeros((3, C * (W + 2), W * C1), dtype=f32)
    m1 = m1.at[:, kk1, nn1].set(
        jnp.broadcast_to(vv1, (3, C, W, 3, C1))).astype(jnp.bfloat16)

    xo2 = jnp.arange(GW)
    kk2 = ((xo2[:, None, None, None] + jnp.arange(3)[None, :, None, None]) * C1
           + jnp.arange(C1)[None, None, :, None])
    nn2 = (xo2[:, None, None, None] * C2
           + jnp.arange(C2)[None, None, None, :])
    vv2 = conv_w2.transpose(2, 3, 1, 0)[:, None, :, :, :]
    m2 = jnp.zeros((3, (GW + 2) * C1, GW * C2), dtype=f32)
    m2 = m2.at[:, kk2, nn2].set(
        jnp.broadcast_to(vv2, (3, GW, 3, C1, C2))).astype(jnp.bfloat16)

    b1t = jnp.tile(conv_b1, W)[None, :]
    b2t = jnp.tile(conv_b2, GW)[None, :]
    whole = lambda s: pl.BlockSpec(s, lambda i: (0,) * len(s))
    grid = (B // TB,)
    gates, load = pl.pallas_call(
        _router_body,
        grid=grid,
        in_specs=[
            pl.BlockSpec((TB, H, C * W), lambda i: (i, 0, 0)),
            pl.BlockSpec((TB, 1), lambda i: (i, 0)),
            whole((3, C * (W + 2), W * C1)),
            whole((3, (GW + 2) * C1, GW * C2)),
            whole((1, W * C1)),
            whole((1, GW * C2)),
            whole((1, D)), whole((1, D)), whole((D, D)), whole((1, D)),
            whole((D, D)), whole((1, D)), whole((D, E)), whole((E, E)),
            whole((1, E)),
        ],
        out_specs=[
            pl.BlockSpec((TB, E), lambda i: (i, 0)),
            pl.BlockSpec((1, E), lambda i: (0, 0)),
        ],
        out_shape=[
            jax.ShapeDtypeStruct((B, E), f32),
            jax.ShapeDtypeStruct((1, E), f32),
        ],
        scratch_shapes=[
            pltpu.VMEM((3, TB, H, C * (W + 2)), jnp.bfloat16),
            pltpu.VMEM((3, TB, H, (W + 2) * C1), jnp.bfloat16),
        ],
    )(xin, tid, m1, m2, b1t, b2t,
      task_w1, task_b1[None, :], task_w2, task_b2[None, :],
      deg_w, deg_b[None, :], expert_keys.T, gate_w, gate_b[None, :])
    return gates, load.reshape(E)
